# baseline (speedup 1.0000x reference)
import functools

import jax
import jax.numpy as jnp
from jax import lax
from jax.experimental import pallas as pl
from jax.experimental.pallas import tpu as pltpu
from jax.experimental.pallas import tpu_sc as plsc

B, C, N, K, OUT = 8, 64, 2048, 20, 64
EPS = 1e-5

R = 256

NCORE, NSUB_CORES, LANES = 2, 16, 16
NW = NCORE * NSUB_CORES
PW = (B * N) // NW
CP = 32
NCHUNK = PW // CP
IPC = CP * K
GSUB = IPC // 128
RPW = (PW * K) // 128
VW = 128


def _tc_body(x_blk_ref, x_full_ref, w_ref, c_ref, idx_ref, u_ref, v_ref):
    b = pl.program_id(0)
    x_blk = x_blk_ref[0]
    x_full = x_full_ref[0]

    inner = lax.dot_general(x_blk, x_full, (((0,), (0,)), ((), ())),
                            preferred_element_type=jnp.float32)
    sq = jnp.sum(x_full * x_full, axis=0, keepdims=True)
    sq_row = jnp.sum(x_blk * x_blk, axis=0)
    d = (sq_row.reshape(R, 1) - 2.0 * inner) + sq

    col = lax.broadcasted_iota(jnp.int32, (R, N), 1)
    big_i = jnp.int32(N)
    inf = jnp.float32(jnp.inf)
    m_prev = jnp.full((R, 1), -inf, jnp.float32)
    c_prev = jnp.full((R, 1), -1, jnp.int32)
    picks = []
    for _ in range(K):
        elig = (d > m_prev) | ((d == m_prev) & (col > c_prev))
        m = jnp.min(jnp.where(elig, d, inf), axis=1, keepdims=True)
        am = jnp.min(jnp.where(elig & (d == m), col, big_i),
                     axis=1, keepdims=True)
        picks.append(am)
        m_prev, c_prev = m, am
    idx_ref[0] = jnp.concatenate(picks, axis=1) + b * N

    w1 = w_ref[:, :C]
    w2 = w_ref[:, C:]
    u_ref[0] = lax.dot_general(x_blk, w1 - w2, (((0,), (1,)), ((), ())),
                               preferred_element_type=jnp.float32) + c_ref[:]
    v = lax.dot_general(x_blk, w2, (((0,), (1,)), ((), ())),
                        preferred_element_type=jnp.float32)
    v_ref[0] = jnp.concatenate(
        [v, jnp.zeros((R, VW - OUT), jnp.float32)], axis=1)


_tc_call = pl.pallas_call(
    _tc_body,
    grid=(B, N // R),
    in_specs=[
        pl.BlockSpec((1, C, R), lambda b, r: (b, 0, r)),
        pl.BlockSpec((1, C, N), lambda b, r: (b, 0, 0)),
        pl.BlockSpec((OUT, 2 * C), lambda b, r: (0, 0)),
        pl.BlockSpec((1, OUT), lambda b, r: (0, 0)),
    ],
    out_specs=[
        pl.BlockSpec((1, R, K), lambda b, r: (b, r, 0)),
        pl.BlockSpec((1, R, OUT), lambda b, r: (b, r, 0)),
        pl.BlockSpec((1, R, VW), lambda b, r: (b, r, 0)),
    ],
    out_shape=[
        jax.ShapeDtypeStruct((B, N, K), jnp.int32),
        jax.ShapeDtypeStruct((B, N, OUT), jnp.float32),
        jax.ShapeDtypeStruct((B, N, VW), jnp.float32),
    ],
    compiler_params=pltpu.CompilerParams(
        dimension_semantics=("parallel", "parallel")),
)


def _sc_gather_max_body(v_hbm, u_hbm, idx_hbm, out_hbm,
                        idx_v, rows_v, u_v, o_v, sem):
    wid = lax.axis_index("s") * NCORE + lax.axis_index("c")
    pltpu.sync_copy(idx_hbm.at[pl.ds(wid * RPW, RPW)], idx_v)

    for ch in range(NCHUNK):
        pbase = wid * PW + ch * CP
        copies = [
            pltpu.async_copy(v_hbm.at[idx_v.at[ch * GSUB + s]],
                             rows_v.at[pl.ds(s * 128, 128)], sem)
            for s in range(GSUB)
        ]
        pltpu.sync_copy(u_hbm.at[pl.ds(pbase, CP)], u_v)
        for cp in copies:
            cp.wait()

        def point_body(p, carry):
            base = p * K
            for cc in range(OUT // LANES):
                sl = pl.ds(cc * LANES, LANES)
                m = rows_v[base, sl]
                for k in range(1, K):
                    m = jnp.maximum(m, rows_v[base + k, sl])
                o_v[p, sl] = jnp.maximum(u_v[p, sl] + m, 0.0)
            return carry

        lax.fori_loop(0, CP, point_body, 0)
        pltpu.sync_copy(o_v, out_hbm.at[pl.ds(pbase, CP)])


@functools.cache
def _sc_gather_max():
    return pl.kernel(
        _sc_gather_max_body,
        mesh=plsc.VectorSubcoreMesh(
            core_axis_name="c", subcore_axis_name="s", num_cores=NCORE),
        out_type=jax.ShapeDtypeStruct((B * N, OUT), jnp.float32),
        scratch_types=[
            pltpu.VMEM((RPW, 128), jnp.int32),
            pltpu.VMEM((IPC, VW), jnp.float32),
            pltpu.VMEM((CP, OUT), jnp.float32),
            pltpu.VMEM((CP, OUT), jnp.float32),
            pltpu.SemaphoreType.DMA,
        ],
    )


def kernel(x, W, b, gamma, beta):
    a = gamma * lax.rsqrt(jnp.float32(1.0) + EPS)
    c = a * b + beta
    Wa = W * a[:, None]
    idx, u, v = _tc_call(x, x, Wa, c.reshape(1, OUT))
    out_pt = _sc_gather_max()(
        v.reshape(B * N, VW),
        u.reshape(B * N, OUT),
        idx.reshape((B * N * K) // 128, 128),
    )
    return out_pt.reshape(B, N, OUT).transpose(0, 2, 1)


# speedup vs baseline: 1.5467x; 1.5467x over previous
import functools

import jax
import jax.numpy as jnp
from jax import lax
from jax.experimental import pallas as pl
from jax.experimental.pallas import tpu as pltpu
from jax.experimental.pallas import tpu_sc as plsc

B, C, N, K, OUT = 8, 64, 2048, 20, 64
EPS = 1e-5

R = 256

NCORE, NSUB_CORES, LANES = 2, 16, 16
NW = NCORE * NSUB_CORES
PW = (B * N) // NW
CP = 32
NCHUNK = PW // CP
IPC = CP * K
GSUB = IPC // 128
RPW = (PW * K) // 128
VW = 128


def _tc_body(x_blk_ref, x_full_ref, w_ref, c_ref, idx_ref, u_ref, v_ref):
    b = pl.program_id(0)
    x_blk = x_blk_ref[0]
    x_full = x_full_ref[0]

    inner = lax.dot_general(x_blk, x_full, (((0,), (0,)), ((), ())),
                            preferred_element_type=jnp.float32)
    sq = jnp.sum(x_full * x_full, axis=0, keepdims=True)
    sq_row = jnp.sum(x_blk * x_blk, axis=0)
    d = (sq_row.reshape(R, 1) - 2.0 * inner) + sq

    col = lax.broadcasted_iota(jnp.int32, (R, N), 1)
    big_i = jnp.int32(N)
    inf = jnp.float32(jnp.inf)
    picks = []
    for _ in range(K):
        m = jnp.min(d, axis=1, keepdims=True)
        am = jnp.min(jnp.where(d == m, col, big_i), axis=1, keepdims=True)
        picks.append(am)
        d = jnp.where(col == am, inf, d)
    idx_ref[0] = jnp.concatenate(picks, axis=1) + b * N

    w1 = w_ref[:, :C]
    w2 = w_ref[:, C:]
    u_ref[0] = lax.dot_general(x_blk, w1 - w2, (((0,), (1,)), ((), ())),
                               preferred_element_type=jnp.float32) + c_ref[:]
    v = lax.dot_general(x_blk, w2, (((0,), (1,)), ((), ())),
                        preferred_element_type=jnp.float32)
    v_ref[0] = jnp.concatenate(
        [v, jnp.zeros((R, VW - OUT), jnp.float32)], axis=1)


_tc_call = pl.pallas_call(
    _tc_body,
    grid=(B, N // R),
    in_specs=[
        pl.BlockSpec((1, C, R), lambda b, r: (b, 0, r)),
        pl.BlockSpec((1, C, N), lambda b, r: (b, 0, 0)),
        pl.BlockSpec((OUT, 2 * C), lambda b, r: (0, 0)),
        pl.BlockSpec((1, OUT), lambda b, r: (0, 0)),
    ],
    out_specs=[
        pl.BlockSpec((1, R, K), lambda b, r: (b, r, 0)),
        pl.BlockSpec((1, R, OUT), lambda b, r: (b, r, 0)),
        pl.BlockSpec((1, R, VW), lambda b, r: (b, r, 0)),
    ],
    out_shape=[
        jax.ShapeDtypeStruct((B, N, K), jnp.int32),
        jax.ShapeDtypeStruct((B, N, OUT), jnp.float32),
        jax.ShapeDtypeStruct((B, N, VW), jnp.float32),
    ],
    compiler_params=pltpu.CompilerParams(
        dimension_semantics=("parallel", "parallel")),
)


def _sc_gather_max_body(v_hbm, u_hbm, idx_hbm, out_hbm,
                        idx_v, rows_v, u_v, o_v, sem):
    wid = lax.axis_index("s") * NCORE + lax.axis_index("c")
    pltpu.sync_copy(idx_hbm.at[pl.ds(wid * RPW, RPW)], idx_v)

    for ch in range(NCHUNK):
        pbase = wid * PW + ch * CP
        copies = [
            pltpu.async_copy(v_hbm.at[idx_v.at[ch * GSUB + s]],
                             rows_v.at[pl.ds(s * 128, 128)], sem)
            for s in range(GSUB)
        ]
        pltpu.sync_copy(u_hbm.at[pl.ds(pbase, CP)], u_v)
        for cp in copies:
            cp.wait()

        def point_body(p, carry):
            base = p * K
            for cc in range(OUT // LANES):
                sl = pl.ds(cc * LANES, LANES)
                m = rows_v[base, sl]
                for k in range(1, K):
                    m = jnp.maximum(m, rows_v[base + k, sl])
                o_v[p, sl] = jnp.maximum(u_v[p, sl] + m, 0.0)
            return carry

        lax.fori_loop(0, CP, point_body, 0)
        pltpu.sync_copy(o_v, out_hbm.at[pl.ds(pbase, CP)])


@functools.cache
def _sc_gather_max():
    return pl.kernel(
        _sc_gather_max_body,
        mesh=plsc.VectorSubcoreMesh(
            core_axis_name="c", subcore_axis_name="s", num_cores=NCORE),
        out_type=jax.ShapeDtypeStruct((B * N, OUT), jnp.float32),
        scratch_types=[
            pltpu.VMEM((RPW, 128), jnp.int32),
            pltpu.VMEM((IPC, VW), jnp.float32),
            pltpu.VMEM((CP, OUT), jnp.float32),
            pltpu.VMEM((CP, OUT), jnp.float32),
            pltpu.SemaphoreType.DMA,
        ],
    )


def kernel(x, W, b, gamma, beta):
    a = gamma * lax.rsqrt(jnp.float32(1.0) + EPS)
    c = a * b + beta
    Wa = W * a[:, None]
    idx, u, v = _tc_call(x, x, Wa, c.reshape(1, OUT))
    out_pt = _sc_gather_max()(
        v.reshape(B * N, VW),
        u.reshape(B * N, OUT),
        idx.reshape((B * N * K) // 128, 128),
    )
    return out_pt.reshape(B, N, OUT).transpose(0, 2, 1)


# speedup vs baseline: 1.9788x; 1.2794x over previous
import functools

import jax
import jax.numpy as jnp
from jax import lax
from jax.experimental import pallas as pl
from jax.experimental.pallas import tpu as pltpu
from jax.experimental.pallas import tpu_sc as plsc

B, C, N, K, OUT = 8, 64, 2048, 20, 64
EPS = 1e-5

R = 256

NCORE, NSUB_CORES, LANES = 2, 16, 16
NW = NCORE * NSUB_CORES
PW = (B * N) // NW
CP = 32
NCHUNK = PW // CP
IPC = CP * K
GSUB = IPC // 128
RPW = (PW * K) // 128
VW = 128


def _tc_body(x_blk_ref, x_full_ref, w_ref, c_ref, idx_ref, u_ref, v_ref):
    b = pl.program_id(0)
    x_blk = x_blk_ref[0]
    x_full = x_full_ref[0]

    inner = lax.dot_general(x_blk, x_full, (((0,), (0,)), ((), ())),
                            preferred_element_type=jnp.float32)
    sq = jnp.sum(x_full * x_full, axis=0, keepdims=True)
    sq_row = jnp.sum(x_blk * x_blk, axis=0)
    d = (sq_row.reshape(R, 1) - 2.0 * inner) + sq

    colf = lax.broadcasted_iota(jnp.int32, (R, N), 1).astype(jnp.float32)
    big_f = jnp.float32(N)
    inf = jnp.float32(jnp.inf)
    picks = []
    for _ in range(K):
        m = jnp.min(d, axis=1, keepdims=True)
        am = jnp.min(jnp.where(d == m, colf, big_f), axis=1, keepdims=True)
        picks.append(am)
        d = jnp.where(colf == am, inf, d)
    idx_ref[0] = (jnp.concatenate(picks, axis=1).astype(jnp.int32)
                  + b * N)

    w1 = w_ref[:, :C]
    w2 = w_ref[:, C:]
    u_ref[0] = lax.dot_general(x_blk, w1 - w2, (((0,), (1,)), ((), ())),
                               preferred_element_type=jnp.float32) + c_ref[:]
    v = lax.dot_general(x_blk, w2, (((0,), (1,)), ((), ())),
                        preferred_element_type=jnp.float32)
    v_ref[0] = jnp.concatenate(
        [v, jnp.zeros((R, VW - OUT), jnp.float32)], axis=1)


_tc_call = pl.pallas_call(
    _tc_body,
    grid=(B, N // R),
    in_specs=[
        pl.BlockSpec((1, C, R), lambda b, r: (b, 0, r)),
        pl.BlockSpec((1, C, N), lambda b, r: (b, 0, 0)),
        pl.BlockSpec((OUT, 2 * C), lambda b, r: (0, 0)),
        pl.BlockSpec((1, OUT), lambda b, r: (0, 0)),
    ],
    out_specs=[
        pl.BlockSpec((1, R, K), lambda b, r: (b, r, 0)),
        pl.BlockSpec((1, R, OUT), lambda b, r: (b, r, 0)),
        pl.BlockSpec((1, R, VW), lambda b, r: (b, r, 0)),
    ],
    out_shape=[
        jax.ShapeDtypeStruct((B, N, K), jnp.int32),
        jax.ShapeDtypeStruct((B, N, OUT), jnp.float32),
        jax.ShapeDtypeStruct((B, N, VW), jnp.float32),
    ],
    compiler_params=pltpu.CompilerParams(
        dimension_semantics=("parallel", "parallel")),
)


def _sc_gather_max_body(v_hbm, u_hbm, idx_hbm, out_hbm,
                        idx_v, rows_v, u_v, o_v, sem):
    wid = lax.axis_index("s") * NCORE + lax.axis_index("c")
    pltpu.sync_copy(idx_hbm.at[pl.ds(wid * RPW, RPW)], idx_v)

    for ch in range(NCHUNK):
        pbase = wid * PW + ch * CP
        copies = [
            pltpu.async_copy(v_hbm.at[idx_v.at[ch * GSUB + s]],
                             rows_v.at[pl.ds(s * 128, 128)], sem)
            for s in range(GSUB)
        ]
        pltpu.sync_copy(u_hbm.at[pl.ds(pbase, CP)], u_v)
        for cp in copies:
            cp.wait()

        def point_body(p, carry):
            base = p * K
            for cc in range(OUT // LANES):
                sl = pl.ds(cc * LANES, LANES)
                m = rows_v[base, sl]
                for k in range(1, K):
                    m = jnp.maximum(m, rows_v[base + k, sl])
                o_v[p, sl] = jnp.maximum(u_v[p, sl] + m, 0.0)
            return carry

        lax.fori_loop(0, CP, point_body, 0)
        pltpu.sync_copy(o_v, out_hbm.at[pl.ds(pbase, CP)])


@functools.cache
def _sc_gather_max():
    return pl.kernel(
        _sc_gather_max_body,
        mesh=plsc.VectorSubcoreMesh(
            core_axis_name="c", subcore_axis_name="s", num_cores=NCORE),
        out_type=jax.ShapeDtypeStruct((B * N, OUT), jnp.float32),
        scratch_types=[
            pltpu.VMEM((RPW, 128), jnp.int32),
            pltpu.VMEM((IPC, VW), jnp.float32),
            pltpu.VMEM((CP, OUT), jnp.float32),
            pltpu.VMEM((CP, OUT), jnp.float32),
            pltpu.SemaphoreType.DMA,
        ],
    )


def kernel(x, W, b, gamma, beta):
    a = gamma * lax.rsqrt(jnp.float32(1.0) + EPS)
    c = a * b + beta
    Wa = W * a[:, None]
    idx, u, v = _tc_call(x, x, Wa, c.reshape(1, OUT))
    out_pt = _sc_gather_max()(
        v.reshape(B * N, VW),
        u.reshape(B * N, OUT),
        idx.reshape((B * N * K) // 128, 128),
    )
    return out_pt.reshape(B, N, OUT).transpose(0, 2, 1)


# speedup vs baseline: 2.0868x; 1.0546x over previous
import functools

import jax
import jax.numpy as jnp
from jax import lax
from jax.experimental import pallas as pl
from jax.experimental.pallas import tpu as pltpu
from jax.experimental.pallas import tpu_sc as plsc

B, C, N, K, OUT = 8, 64, 2048, 20, 64
EPS = 1e-5

R = 256
HB = 4

NCORE, NSUB_CORES, LANES = 2, 16, 16
NW = NCORE * NSUB_CORES
PW = (HB * N) // NW
CP = 32
NCHUNK = PW // CP
IPC = CP * K
GSUB = IPC // 128
RPW = (PW * K) // 128
VW = 128


def _tc_body(x_blk_ref, x_full_ref, w_ref, c_ref, idx_ref, u_ref, v_ref):
    b = pl.program_id(0)
    x_blk = x_blk_ref[0]
    x_full = x_full_ref[0]

    inner = lax.dot_general(x_blk, x_full, (((0,), (0,)), ((), ())),
                            preferred_element_type=jnp.float32)
    sq = jnp.sum(x_full * x_full, axis=0, keepdims=True)
    sq_row = jnp.sum(x_blk * x_blk, axis=0)
    d = (sq_row.reshape(R, 1) - 2.0 * inner) + sq

    colf = lax.broadcasted_iota(jnp.int32, (R, N), 1).astype(jnp.float32)
    big_f = jnp.float32(N)
    inf = jnp.float32(jnp.inf)
    picks = []
    for _ in range(K):
        m = jnp.min(d, axis=1, keepdims=True)
        am = jnp.min(jnp.where(d == m, colf, big_f), axis=1, keepdims=True)
        picks.append(am)
        d = jnp.where(colf == am, inf, d)
    idx_ref[0] = (jnp.concatenate(picks, axis=1).astype(jnp.int32)
                  + b * N)

    w1 = w_ref[:, :C]
    w2 = w_ref[:, C:]
    u_ref[0] = lax.dot_general(x_blk, w1 - w2, (((0,), (1,)), ((), ())),
                               preferred_element_type=jnp.float32) + c_ref[:]
    v = lax.dot_general(x_blk, w2, (((0,), (1,)), ((), ())),
                        preferred_element_type=jnp.float32)
    v_ref[0] = jnp.concatenate(
        [v, jnp.zeros((R, VW - OUT), jnp.float32)], axis=1)


_tc_call = pl.pallas_call(
    _tc_body,
    grid=(HB, N // R),
    in_specs=[
        pl.BlockSpec((1, C, R), lambda b, r: (b, 0, r)),
        pl.BlockSpec((1, C, N), lambda b, r: (b, 0, 0)),
        pl.BlockSpec((OUT, 2 * C), lambda b, r: (0, 0)),
        pl.BlockSpec((1, OUT), lambda b, r: (0, 0)),
    ],
    out_specs=[
        pl.BlockSpec((1, R, K), lambda b, r: (b, r, 0)),
        pl.BlockSpec((1, R, OUT), lambda b, r: (b, r, 0)),
        pl.BlockSpec((1, R, VW), lambda b, r: (b, r, 0)),
    ],
    out_shape=[
        jax.ShapeDtypeStruct((HB, N, K), jnp.int32),
        jax.ShapeDtypeStruct((HB, N, OUT), jnp.float32),
        jax.ShapeDtypeStruct((HB, N, VW), jnp.float32),
    ],
    compiler_params=pltpu.CompilerParams(
        dimension_semantics=("parallel", "parallel")),
)


def _sc_gather_max_body(v_hbm, u_hbm, idx_hbm, out_hbm,
                        idx_v, rows_v, u_v, o_v, sem):
    wid = lax.axis_index("s") * NCORE + lax.axis_index("c")
    pltpu.sync_copy(idx_hbm.at[pl.ds(wid * RPW, RPW)], idx_v)

    for ch in range(NCHUNK):
        pbase = wid * PW + ch * CP
        copies = [
            pltpu.async_copy(v_hbm.at[idx_v.at[ch * GSUB + s]],
                             rows_v.at[pl.ds(s * 128, 128)], sem)
            for s in range(GSUB)
        ]
        pltpu.sync_copy(u_hbm.at[pl.ds(pbase, CP)], u_v)
        for cp in copies:
            cp.wait()

        def point_body(p, carry):
            base = p * K
            for cc in range(OUT // LANES):
                sl = pl.ds(cc * LANES, LANES)
                m = rows_v[base, sl]
                for k in range(1, K):
                    m = jnp.maximum(m, rows_v[base + k, sl])
                o_v[p, sl] = jnp.maximum(u_v[p, sl] + m, 0.0)
            return carry

        lax.fori_loop(0, CP, point_body, 0)
        pltpu.sync_copy(o_v, out_hbm.at[pl.ds(pbase, CP)])


@functools.cache
def _sc_gather_max():
    return pl.kernel(
        _sc_gather_max_body,
        mesh=plsc.VectorSubcoreMesh(
            core_axis_name="c", subcore_axis_name="s", num_cores=NCORE),
        out_type=jax.ShapeDtypeStruct((HB * N, OUT), jnp.float32),
        scratch_types=[
            pltpu.VMEM((RPW, 128), jnp.int32),
            pltpu.VMEM((IPC, VW), jnp.float32),
            pltpu.VMEM((CP, OUT), jnp.float32),
            pltpu.VMEM((CP, OUT), jnp.float32),
            pltpu.SemaphoreType.DMA,
        ],
    )


def kernel(x, W, b, gamma, beta):
    a = gamma * lax.rsqrt(jnp.float32(1.0) + EPS)
    c = a * b + beta
    Wa = W * a[:, None]
    c2 = c.reshape(1, OUT)
    sc = _sc_gather_max()
    outs = []
    for h in range(B // HB):
        xh = x[h * HB:(h + 1) * HB]
        idx, u, v = _tc_call(xh, xh, Wa, c2)
        outs.append(sc(
            v.reshape(HB * N, VW),
            u.reshape(HB * N, OUT),
            idx.reshape((HB * N * K) // 128, 128),
        ))
    out_pt = jnp.concatenate(outs, axis=0)
    return out_pt.reshape(B, N, OUT).transpose(0, 2, 1)


# speedup vs baseline: 2.1395x; 1.0253x over previous
import functools

import jax
import jax.numpy as jnp
from jax import lax
from jax.experimental import pallas as pl
from jax.experimental.pallas import tpu as pltpu
from jax.experimental.pallas import tpu_sc as plsc

B, C, N, K, OUT = 8, 64, 2048, 20, 64
EPS = 1e-5

R = 256
HB = 2

NCORE, NSUB_CORES, LANES = 2, 16, 16
NW = NCORE * NSUB_CORES
PW = (HB * N) // NW
CP = 32
NCHUNK = PW // CP
IPC = CP * K
GSUB = IPC // 128
RPW = (PW * K) // 128
VW = 128


def _tc_body(x_blk_ref, x_full_ref, w_ref, c_ref, idx_ref, u_ref, v_ref):
    b = pl.program_id(0)
    x_blk = x_blk_ref[0]
    x_full = x_full_ref[0]

    inner = lax.dot_general(x_blk, x_full, (((0,), (0,)), ((), ())),
                            preferred_element_type=jnp.float32)
    sq = jnp.sum(x_full * x_full, axis=0, keepdims=True)
    sq_row = jnp.sum(x_blk * x_blk, axis=0)
    d = (sq_row.reshape(R, 1) - 2.0 * inner) + sq

    colf = lax.broadcasted_iota(jnp.int32, (R, N), 1).astype(jnp.float32)
    big_f = jnp.float32(N)
    inf = jnp.float32(jnp.inf)
    picks = []
    for _ in range(K):
        m = jnp.min(d, axis=1, keepdims=True)
        am = jnp.min(jnp.where(d == m, colf, big_f), axis=1, keepdims=True)
        picks.append(am)
        d = jnp.where(colf == am, inf, d)
    idx_ref[0] = (jnp.concatenate(picks, axis=1).astype(jnp.int32)
                  + b * N)

    w1 = w_ref[:, :C]
    w2 = w_ref[:, C:]
    u_ref[0] = lax.dot_general(x_blk, w1 - w2, (((0,), (1,)), ((), ())),
                               preferred_element_type=jnp.float32) + c_ref[:]
    v = lax.dot_general(x_blk, w2, (((0,), (1,)), ((), ())),
                        preferred_element_type=jnp.float32)
    v_ref[0] = jnp.concatenate(
        [v, jnp.zeros((R, VW - OUT), jnp.float32)], axis=1)


_tc_call = pl.pallas_call(
    _tc_body,
    grid=(HB, N // R),
    in_specs=[
        pl.BlockSpec((1, C, R), lambda b, r: (b, 0, r)),
        pl.BlockSpec((1, C, N), lambda b, r: (b, 0, 0)),
        pl.BlockSpec((OUT, 2 * C), lambda b, r: (0, 0)),
        pl.BlockSpec((1, OUT), lambda b, r: (0, 0)),
    ],
    out_specs=[
        pl.BlockSpec((1, R, K), lambda b, r: (b, r, 0)),
        pl.BlockSpec((1, R, OUT), lambda b, r: (b, r, 0)),
        pl.BlockSpec((1, R, VW), lambda b, r: (b, r, 0)),
    ],
    out_shape=[
        jax.ShapeDtypeStruct((HB, N, K), jnp.int32),
        jax.ShapeDtypeStruct((HB, N, OUT), jnp.float32),
        jax.ShapeDtypeStruct((HB, N, VW), jnp.float32),
    ],
    compiler_params=pltpu.CompilerParams(
        dimension_semantics=("parallel", "parallel")),
)


def _sc_gather_max_body(v_hbm, u_hbm, idx_hbm, out_hbm,
                        idx_v, rows_v, u_v, o_v, sem):
    wid = lax.axis_index("s") * NCORE + lax.axis_index("c")
    pltpu.sync_copy(idx_hbm.at[wid], idx_v)

    for ch in range(NCHUNK):
        pbase = wid * PW + ch * CP
        copies = [
            pltpu.async_copy(v_hbm.at[idx_v.at[ch * GSUB + s]],
                             rows_v.at[pl.ds(s * 128, 128)], sem)
            for s in range(GSUB)
        ]
        pltpu.sync_copy(u_hbm.at[pl.ds(pbase, CP)], u_v)
        for cp in copies:
            cp.wait()

        def point_body(p, carry):
            base = p * K
            for cc in range(OUT // LANES):
                sl = pl.ds(cc * LANES, LANES)
                m = rows_v[base, sl]
                for k in range(1, K):
                    m = jnp.maximum(m, rows_v[base + k, sl])
                o_v[p, sl] = jnp.maximum(u_v[p, sl] + m, 0.0)
            return carry

        lax.fori_loop(0, CP, point_body, 0)
        pltpu.sync_copy(o_v, out_hbm.at[pl.ds(pbase, CP)])


@functools.cache
def _sc_gather_max():
    return pl.kernel(
        _sc_gather_max_body,
        mesh=plsc.VectorSubcoreMesh(
            core_axis_name="c", subcore_axis_name="s", num_cores=NCORE),
        out_type=jax.ShapeDtypeStruct((HB * N, OUT), jnp.float32),
        scratch_types=[
            pltpu.VMEM((RPW, 128), jnp.int32),
            pltpu.VMEM((IPC, VW), jnp.float32),
            pltpu.VMEM((CP, OUT), jnp.float32),
            pltpu.VMEM((CP, OUT), jnp.float32),
            pltpu.SemaphoreType.DMA,
        ],
    )


def kernel(x, W, b, gamma, beta):
    a = gamma * lax.rsqrt(jnp.float32(1.0) + EPS)
    c = a * b + beta
    Wa = W * a[:, None]
    c2 = c.reshape(1, OUT)
    sc = _sc_gather_max()
    outs = []
    for h in range(B // HB):
        xh = x[h * HB:(h + 1) * HB]
        idx, u, v = _tc_call(xh, xh, Wa, c2)
        outs.append(sc(
            v.reshape(HB * N, VW),
            u.reshape(HB * N, OUT),
            idx.reshape(NW, RPW, 128),
        ))
    out_pt = jnp.concatenate(outs, axis=0)
    return out_pt.reshape(B, N, OUT).transpose(0, 2, 1)
